# baseline (speedup 1.0000x reference)
import functools
import math

import jax
import jax.numpy as jnp
from jax import lax
from jax.experimental import pallas as pl
from jax.experimental.pallas import tpu as pltpu
from jax.experimental.pallas import tpu_sc as plsc

B, L, D, V = 16384, 200, 32, 100000
N = B * L
NUM_WORKERS = 32
UNIT = 128
NU = N // UNIT
UPT = NU // NUM_WORKERS
NB = B // UNIT
TD = D // 8
SCALE = math.sqrt(float(D))


_SCALE_GRID = 20
_SCALE_ROWS = V // _SCALE_GRID


def _prescale_body(t_ref, o_ref):
    o_ref[...] = t_ref[...] * SCALE

    @pl.when(pl.program_id(0) == 0)
    def _zero_pad_row():
        o_ref[0:1, :] = jnp.zeros((1, D), jnp.float32)


_prescale = pl.pallas_call(
    _prescale_body,
    grid=(_SCALE_GRID,),
    in_specs=[pl.BlockSpec((_SCALE_ROWS, D), lambda i: (i, 0))],
    out_specs=pl.BlockSpec((_SCALE_ROWS, D), lambda i: (i, 0)),
    out_shape=jax.ShapeDtypeStruct((V, D), jnp.float32),
)


_mesh = plsc.VectorSubcoreMesh(core_axis_name="c", subcore_axis_name="s")


@functools.partial(
    pl.kernel,
    mesh=_mesh,
    compiler_params=pltpu.CompilerParams(use_tc_tiling_on_sc=False,
                                         needs_layout_passes=False),
    out_type=jax.ShapeDtypeStruct((L, TD, NB, 8, UNIT), jnp.float32),
    scratch_types=[
        pltpu.VMEM((4, UNIT), jnp.int32),
        pltpu.VMEM((4, UNIT, D), jnp.float32),
        pltpu.VMEM((4, D, UNIT + 1), jnp.float32),
        pltpu.SemaphoreType.DMA,
        pltpu.SemaphoreType.DMA,
        pltpu.SemaphoreType.DMA,
        pltpu.SemaphoreType.DMA,
        pltpu.SemaphoreType.DMA,
        pltpu.SemaphoreType.DMA,
        pltpu.SemaphoreType.DMA,
        pltpu.SemaphoreType.DMA,
        pltpu.SemaphoreType.DMA,
        pltpu.SemaphoreType.DMA,
        pltpu.SemaphoreType.DMA,
        pltpu.SemaphoreType.DMA,
    ],
)
def _gather(tbl_hbm, idx_hbm, out_hbm, idx_v, rows_v, tbuf,
            si0, si1, si2, si3, sg0, sg1, sg2, sg3, so0, so1, so2, so3):
    sem_i = (si0, si1, si2, si3)
    sem_g = (sg0, sg1, sg2, sg3)
    sem_o = (so0, so1, so2, so3)
    wid = lax.axis_index("s") * 2 + lax.axis_index("c")
    u0 = wid * UPT

    row16 = lax.iota(jnp.int32, 16)

    def start_idx(u, b):
        pltpu.async_copy(idx_hbm.at[pl.ds((u0 + u) * UNIT, UNIT)],
                         idx_v.at[b], sem_i[b])

    def wait_idx(u, b):
        pltpu.make_async_copy(idx_hbm.at[pl.ds(u0 * UNIT, UNIT)],
                              idx_v.at[b], sem_i[b]).wait()

    def start_gather(b):
        pltpu.async_copy(tbl_hbm.at[idx_v.at[b]], rows_v.at[b], sem_g[b])

    def wait_gather(b):
        pltpu.make_async_copy(tbl_hbm.at[idx_v.at[b]], rows_v.at[b],
                              sem_g[b]).wait()

    def start_out(u, b, tds=tuple(range(TD))):
        ug = u0 + u
        l = ug // NB
        blk = lax.rem(ug, NB)
        for td in tds:
            pltpu.async_copy(tbuf.at[b, pl.ds(8 * td, 8), pl.ds(0, UNIT)],
                             out_hbm.at[l, td, blk], sem_o[b])

    def wait_out(b):
        for td in range(TD):
            pltpu.make_async_copy(tbuf.at[b, pl.ds(8 * td, 8), pl.ds(0, UNIT)],
                                  out_hbm.at[0, 0, 0], sem_o[b]).wait()

    for b in range(4):
        start_idx(b, b)
    for b in range(2):
        wait_idx(b, b)
        start_gather(b)

    def body(i, carry):
        for b in range(4):
            u = 4 * i + b
            wait_gather(b)

            @pl.when(u + 4 < UPT)
            def _prefetch_idx():
                start_idx(u + 4, b)

            b2 = (b + 2) % 4

            @pl.when(jnp.logical_and(u >= 2, u + 2 < UPT))
            def _reclaim_rows():
                wait_out(b2)

            @pl.when(u + 2 < UPT)
            def _next_gather():
                wait_idx(u + 2, b2)
                start_gather(b2)

            for f in range(2):
                def jloop(jj, colj, f=f):
                    for r in range(8):
                        j = 8 * jj + r
                        vec = rows_v[b, j, pl.ds(16 * f, 16)]
                        plsc.store_scatter(tbuf.at[b],
                                           [row16 + (16 * f), colj], vec)
                        colj = colj + 1
                    return colj

                lax.fori_loop(0, UNIT // 8, jloop,
                              jnp.zeros((16,), jnp.int32))
                start_out(u, b, (2 * f, 2 * f + 1))
        return carry

    lax.fori_loop(0, UPT // 4, body, 0)

    for b in range(4):
        wait_out(b)


def kernel(inputs, table):
    idx = jnp.transpose(inputs).reshape(-1).astype(jnp.int32)
    tbl2 = _prescale(table)
    a5 = _gather(tbl2, idx)
    return a5.transpose(2, 4, 0, 1, 3).reshape(B, L, D)


# speedup vs baseline: 1.0213x; 1.0213x over previous
import functools
import math

import jax
import jax.numpy as jnp
from jax import lax
from jax.experimental import pallas as pl
from jax.experimental.pallas import tpu as pltpu
from jax.experimental.pallas import tpu_sc as plsc

B, L, D, V = 16384, 200, 32, 100000
N = B * L
NUM_WORKERS = 32
UNIT = 128
NU = N // UNIT
UPT = NU // NUM_WORKERS
NB = B // UNIT
TD = D // 8
SCALE = math.sqrt(float(D))


_SCALE_GRID = 20
_SCALE_ROWS = V // _SCALE_GRID


def _prescale_body(t_ref, o_ref):
    o_ref[...] = t_ref[...] * SCALE

    @pl.when(pl.program_id(0) == 0)
    def _zero_pad_row():
        o_ref[0:1, :] = jnp.zeros((1, D), jnp.float32)


_prescale = pl.pallas_call(
    _prescale_body,
    grid=(_SCALE_GRID,),
    in_specs=[pl.BlockSpec((_SCALE_ROWS, D), lambda i: (i, 0))],
    out_specs=pl.BlockSpec((_SCALE_ROWS, D), lambda i: (i, 0)),
    out_shape=jax.ShapeDtypeStruct((V, D), jnp.float32),
)


_mesh = plsc.VectorSubcoreMesh(core_axis_name="c", subcore_axis_name="s")


@functools.partial(
    pl.kernel,
    mesh=_mesh,
    compiler_params=pltpu.CompilerParams(use_tc_tiling_on_sc=False,
                                         needs_layout_passes=False),
    out_type=jax.ShapeDtypeStruct((L, TD, NB, 8, UNIT), jnp.float32),
    scratch_types=[
        pltpu.VMEM((4, UNIT), jnp.int32),
        pltpu.VMEM((4, UNIT, D), jnp.float32),
        pltpu.VMEM((4, D, UNIT + 1), jnp.float32),
        pltpu.SemaphoreType.DMA,
        pltpu.SemaphoreType.DMA,
        pltpu.SemaphoreType.DMA,
        pltpu.SemaphoreType.DMA,
        pltpu.SemaphoreType.DMA,
        pltpu.SemaphoreType.DMA,
        pltpu.SemaphoreType.DMA,
        pltpu.SemaphoreType.DMA,
        pltpu.SemaphoreType.DMA,
        pltpu.SemaphoreType.DMA,
        pltpu.SemaphoreType.DMA,
        pltpu.SemaphoreType.DMA,
    ],
)
def _gather(tbl_hbm, idx_hbm, out_hbm, idx_v, rows_v, tbuf,
            si0, si1, si2, si3, sg0, sg1, sg2, sg3, so0, so1, so2, so3):
    sem_i = (si0, si1, si2, si3)
    sem_g = (sg0, sg1, sg2, sg3)
    sem_o = (so0, so1, so2, so3)
    wid = lax.axis_index("s") * 2 + lax.axis_index("c")
    u0 = wid * UPT

    row16 = lax.iota(jnp.int32, 16)

    def start_idx(u, b):
        pltpu.async_copy(idx_hbm.at[pl.ds((u0 + u) * UNIT, UNIT)],
                         idx_v.at[b], sem_i[b])

    def wait_idx(u, b):
        pltpu.make_async_copy(idx_hbm.at[pl.ds(u0 * UNIT, UNIT)],
                              idx_v.at[b], sem_i[b]).wait()

    def start_gather(b):
        pltpu.async_copy(tbl_hbm.at[idx_v.at[b]], rows_v.at[b], sem_g[b])

    def wait_gather(b):
        pltpu.make_async_copy(tbl_hbm.at[idx_v.at[b]], rows_v.at[b],
                              sem_g[b]).wait()

    def start_out(u, b):
        ug = u0 + u
        l = ug // NB
        blk = lax.rem(ug, NB)
        for td in range(TD):
            pltpu.async_copy(tbuf.at[b, pl.ds(8 * td, 8), pl.ds(0, UNIT)],
                             out_hbm.at[l, td, blk], sem_o[b])

    def wait_out(b):
        for td in range(TD):
            pltpu.make_async_copy(tbuf.at[b, pl.ds(8 * td, 8), pl.ds(0, UNIT)],
                                  out_hbm.at[0, 0, 0], sem_o[b]).wait()

    for b in range(4):
        start_idx(b, b)
    for b in range(2):
        wait_idx(b, b)
        start_gather(b)

    def body(i, carry):
        for b in range(4):
            u = 4 * i + b
            wait_gather(b)

            @pl.when(u + 4 < UPT)
            def _prefetch_idx():
                start_idx(u + 4, b)

            b2 = (b + 2) % 4

            @pl.when(jnp.logical_and(u >= 2, u + 2 < UPT))
            def _reclaim_rows():
                wait_out(b2)

            @pl.when(u + 2 < UPT)
            def _next_gather():
                wait_idx(u + 2, b2)
                start_gather(b2)

            def jloop(jj, c):
                for r in range(8):
                    j = 8 * jj + r
                    colj = jnp.full((16,), j, jnp.int32)
                    for f in range(2):
                        vec = rows_v[b, j, pl.ds(16 * f, 16)]
                        plsc.store_scatter(tbuf.at[b],
                                           [row16 + (16 * f), colj], vec)
                return c

            lax.fori_loop(0, UNIT // 8, jloop, 0)

            start_out(u, b)
        return carry

    lax.fori_loop(0, UPT // 4, body, 0)

    for b in range(4):
        wait_out(b)


def kernel(inputs, table):
    idx = jnp.transpose(inputs).reshape(-1).astype(jnp.int32)
    tbl2 = _prescale(table)
    a5 = _gather(tbl2, idx)
    return a5.transpose(2, 4, 0, 1, 3).reshape(B, L, D)
